# baseline (speedup 1.0000x reference)
import jax
import jax.numpy as jnp
from jax.experimental import pallas as pl

_K = 1024
_D = 64
_B = 16
_H = 32
_W = 32
_BETA = 0.25
_HW = _H * _W
_N = _B * _HW


def _vq_body(z_ref, cb_ref, zq_ref, idx_ref, loss_ref):
    zb = z_ref[0].reshape(_D, _HW)
    cb = cb_ref[...]
    z2 = jnp.sum(zb * zb, axis=0, keepdims=True)
    c2 = jnp.sum(cb * cb, axis=1, keepdims=True)
    s2 = jax.lax.dot_general(
        cb, zb + zb, (((1,), (0,)), ((), ())),
        preferred_element_type=jnp.float32)
    d = (z2 + c2) - s2
    dmin = jnp.min(d, axis=0, keepdims=True)
    kio = jax.lax.broadcasted_iota(jnp.int32, d.shape, 0).astype(jnp.float32)
    idxf = jnp.min(jnp.where(d == dmin, kio, float(_K)), axis=0, keepdims=True)
    oh = (kio == idxf).astype(jnp.float32)
    zq = jax.lax.dot_general(
        cb, oh, (((0,), (0,)), ((), ())),
        preferred_element_type=jnp.float32)
    zq_ref[0] = (zb + (zq - zb)).reshape(_D, _H, _W)
    i = pl.program_id(0)
    idx_ref[pl.ds(i, 1), :] = idxf.astype(jnp.int32)
    part = jnp.sum((zq - zb) ** 2).reshape(1, 1)

    @pl.when(i == 0)
    def _init():
        loss_ref[...] = jnp.zeros((1, 1), jnp.float32)

    loss_ref[...] += part

    @pl.when(i == _B - 1)
    def _finish():
        loss_ref[...] = loss_ref[...] * ((1.0 + _BETA) / float(_N * _D))


def kernel(z, codebook):
    Bz, Dz, Hz, Wz = z.shape
    z_q_st, indices, loss11 = pl.pallas_call(
        _vq_body,
        grid=(_B,),
        in_specs=[
            pl.BlockSpec((1, _D, _H, _W), lambda i: (i, 0, 0, 0)),
            pl.BlockSpec((_K, _D), lambda i: (0, 0)),
        ],
        out_specs=[
            pl.BlockSpec((1, _D, _H, _W), lambda i: (i, 0, 0, 0)),
            pl.BlockSpec((_B, _HW), lambda i: (0, 0)),
            pl.BlockSpec((1, 1), lambda i: (0, 0)),
        ],
        out_shape=[
            jax.ShapeDtypeStruct((_B, _D, _H, _W), jnp.float32),
            jax.ShapeDtypeStruct((_B, _HW), jnp.int32),
            jax.ShapeDtypeStruct((1, 1), jnp.float32),
        ],
    )(z, codebook)
    return (z_q_st, loss11.reshape(()), indices)


# speedup vs baseline: 1.6167x; 1.6167x over previous
import jax
import jax.numpy as jnp
from jax.experimental import pallas as pl

_K = 1024
_D = 64
_B = 16
_H = 32
_W = 32
_BETA = 0.25
_HW = _H * _W
_N = _B * _HW


def _vq_body(z_ref, cb_ref, zq_ref, idx_ref, loss_ref):
    zb = z_ref[0]
    cb = cb_ref[...]
    z2 = jnp.sum(zb * zb, axis=0, keepdims=True)
    c2 = jnp.sum(cb * cb, axis=1, keepdims=True)
    s2 = jax.lax.dot_general(
        cb, zb + zb, (((1,), (0,)), ((), ())),
        preferred_element_type=jnp.float32)
    d = (z2 + c2) - s2
    idx = jnp.argmin(d, axis=0)
    kio = jax.lax.broadcasted_iota(jnp.int32, d.shape, 0)
    oh = (kio == idx[None, :]).astype(jnp.float32)
    zq = jax.lax.dot_general(
        cb, oh, (((0,), (0,)), ((), ())),
        preferred_element_type=jnp.float32)
    zq_ref[0] = zb + (zq - zb)
    i = pl.program_id(0)
    idx_ref[pl.ds(i, 1), :] = idx.reshape(1, _HW)
    part = jnp.sum((zq - zb) ** 2).reshape(1, 1)

    @pl.when(i == 0)
    def _init():
        loss_ref[...] = jnp.zeros((1, 1), jnp.float32)

    loss_ref[...] += part

    @pl.when(i == _B - 1)
    def _finish():
        loss_ref[...] = loss_ref[...] * ((1.0 + _BETA) / float(_N * _D))


def kernel(z, codebook):
    Bz, Dz, Hz, Wz = z.shape
    z3 = z.reshape(Bz, Dz, Hz * Wz)
    zq3, indices, loss11 = pl.pallas_call(
        _vq_body,
        grid=(_B,),
        in_specs=[
            pl.BlockSpec((1, _D, _HW), lambda i: (i, 0, 0)),
            pl.BlockSpec((_K, _D), lambda i: (0, 0)),
        ],
        out_specs=[
            pl.BlockSpec((1, _D, _HW), lambda i: (i, 0, 0)),
            pl.BlockSpec((_B, _HW), lambda i: (0, 0)),
            pl.BlockSpec((1, 1), lambda i: (0, 0)),
        ],
        out_shape=[
            jax.ShapeDtypeStruct((_B, _D, _HW), jnp.float32),
            jax.ShapeDtypeStruct((_B, _HW), jnp.int32),
            jax.ShapeDtypeStruct((1, 1), jnp.float32),
        ],
    )(z3, codebook)
    z_q_st = zq3.reshape(Bz, Dz, Hz, Wz)
    return (z_q_st, loss11[0, 0], indices)
